# baseline (speedup 1.0000x reference)
import functools

import jax
import jax.numpy as jnp
from jax import lax
from jax.experimental import pallas as pl
from jax.experimental.pallas import tpu as pltpu
from jax.experimental.pallas import tpu_sc as plsc

_NW = 32
_CH = 128
_SCH = 32
_NB = 4
_NSPLIT = 1


def _sc_gather(emb, nbr2f, sel, split, nsplit):
    V, D = emb.shape
    N2 = nbr2f.shape[0]
    NP = N2 // nsplit
    n2_w = NP // _NW
    n2_ch = n2_w // _CH
    gbase = split * NP

    out_type = [jax.ShapeDtypeStruct((NP, D), emb.dtype)]
    scratch = [
        pltpu.VMEM((_NB, _CH), jnp.int32),
        pltpu.VMEM((_NB, _CH, D), emb.dtype),
        pltpu.SemaphoreType.DMA((_NB,)),
        pltpu.SemaphoreType.DMA((_NB,)),
    ]
    if sel is not None:
        NS = sel.shape[0]
        sel_w = NS // _NW
        sel_ch = sel_w // _SCH
        out_type.append(jax.ShapeDtypeStruct((NS, D), emb.dtype))
        scratch += [
            pltpu.VMEM((2, _SCH), jnp.int32),
            pltpu.VMEM((2, _SCH, D), emb.dtype),
            pltpu.SemaphoreType.DMA((2,)),
            pltpu.SemaphoreType.DMA((2,)),
        ]

    mesh = plsc.VectorSubcoreMesh(core_axis_name="c", subcore_axis_name="s")

    @functools.partial(pl.kernel, mesh=mesh, out_type=out_type,
                       scratch_types=scratch)
    def k(*refs):
        if sel is not None:
            (emb_hbm, n2_hbm, sel_hbm, h2_hbm, hs_hbm,
             idx_v, rows_v, gs, ws, sidx_v, srows_v, sgs, sws) = refs
        else:
            emb_hbm, n2_hbm, h2_hbm, idx_v, rows_v, gs, ws = refs

        wid = lax.axis_index("s") * 2 + lax.axis_index("c")
        base2 = wid * n2_w

        @pl.loop(0, n2_ch, step=_NB)
        def _(c):
            for b in range(_NB):
                cc = c + b

                @pl.when(cc >= _NB)
                def _():
                    pltpu.make_async_copy(
                        rows_v.at[b], h2_hbm.at[pl.ds(0, _CH)], ws.at[b]
                    ).wait()

                pltpu.sync_copy(
                    n2_hbm.at[pl.ds(gbase + base2 + cc * _CH, _CH)],
                    idx_v.at[b])
                pltpu.async_copy(emb_hbm.at[idx_v.at[b]], rows_v.at[b],
                                 gs.at[b])
                pb = (b - 1) % _NB

                @pl.when(cc >= 1)
                def _():
                    pltpu.make_async_copy(
                        emb_hbm.at[idx_v.at[pb]], rows_v.at[pb], gs.at[pb]
                    ).wait()
                    pltpu.async_copy(rows_v.at[pb],
                                     h2_hbm.at[pl.ds(base2 + cc * _CH - _CH,
                                                     _CH)],
                                     ws.at[pb])

        lb = (n2_ch - 1) % _NB
        pltpu.make_async_copy(emb_hbm.at[idx_v.at[lb]], rows_v.at[lb],
                              gs.at[lb]).wait()
        pltpu.async_copy(rows_v.at[lb],
                         h2_hbm.at[pl.ds(base2 + (n2_ch - 1) * _CH, _CH)],
                         ws.at[lb])
        for b in range(_NB):
            pltpu.make_async_copy(rows_v.at[b], h2_hbm.at[pl.ds(0, _CH)],
                                  ws.at[b]).wait()

        if sel is not None:
            bases = wid * sel_w

            @pl.loop(0, sel_ch, step=2)
            def _(c):
                for b in range(2):
                    cc = c + b

                    @pl.when(cc >= 2)
                    def _():
                        pltpu.make_async_copy(
                            srows_v.at[b], hs_hbm.at[pl.ds(0, _SCH)],
                            sws.at[b]).wait()

                    pltpu.sync_copy(sel_hbm.at[pl.ds(bases + cc * _SCH,
                                                     _SCH)],
                                    sidx_v.at[b])
                    pltpu.async_copy(emb_hbm.at[sidx_v.at[b]], srows_v.at[b],
                                     sgs.at[b])
                    pb = 1 - b

                    @pl.when(cc >= 1)
                    def _():
                        pltpu.make_async_copy(
                            emb_hbm.at[sidx_v.at[pb]], srows_v.at[pb],
                            sgs.at[pb]).wait()
                        pltpu.async_copy(
                            srows_v.at[pb],
                            hs_hbm.at[pl.ds(bases + cc * _SCH - _SCH, _SCH)],
                            sws.at[pb])

            lbs = (sel_ch - 1) % 2
            pltpu.make_async_copy(emb_hbm.at[sidx_v.at[lbs]],
                                  srows_v.at[lbs], sgs.at[lbs]).wait()
            pltpu.async_copy(srows_v.at[lbs],
                             hs_hbm.at[pl.ds(bases + (sel_ch - 1) * _SCH,
                                             _SCH)],
                             sws.at[lbs])
            for b in range(2):
                pltpu.make_async_copy(srows_v.at[b],
                                      hs_hbm.at[pl.ds(0, _SCH)],
                                      sws.at[b]).wait()

    if sel is not None:
        return k(emb, nbr2f, sel)
    res = k(emb, nbr2f)
    return res[0] if isinstance(res, (list, tuple)) else res


def _tc_dense(h2part, hsel, G1T, W1s, W1n, W2s, W2n, split, nsplit,
              B, F1, F2, D, H):
    BLK = 512
    OB = BLK // F1
    nblk = (B * F1) // BLK // nsplit
    h1_blk0 = split * nblk
    h0_blk0 = (B * F1) // OB + split * ((B // nsplit) // OB)

    def body(h2v_ref, hs_ref, h0_ref, g_ref, w1s_ref, w1n_ref, w2s_ref,
             w2n_ref, o_ref):
        h2m = jnp.sum(h2v_ref[...].reshape(BLK, F2, D), axis=1) * (1.0 / F2)
        h1 = jnp.maximum(
            jnp.dot(hs_ref[...], w1s_ref[...],
                    preferred_element_type=jnp.float32)
            + jnp.dot(h2m, w1n_ref[...], preferred_element_type=jnp.float32),
            0.0)
        h1m = jnp.dot(g_ref[...], h1, preferred_element_type=jnp.float32)
        o_ref[...] = jnp.maximum(
            jnp.dot(h0_ref[...], w2s_ref[...],
                    preferred_element_type=jnp.float32)
            + jnp.dot(h1m, w2n_ref[...], preferred_element_type=jnp.float32),
            0.0)

    return pl.pallas_call(
        body,
        grid=(nblk,),
        in_specs=[
            pl.BlockSpec((BLK * F2, D), lambda i: (i, 0)),
            pl.BlockSpec((BLK, D), lambda i: (i + h1_blk0, 0)),
            pl.BlockSpec((OB, D), lambda i: (i + h0_blk0, 0)),
            pl.BlockSpec((OB, BLK), lambda i: (0, 0)),
            pl.BlockSpec((D, H), lambda i: (0, 0)),
            pl.BlockSpec((D, H), lambda i: (0, 0)),
            pl.BlockSpec((D, H), lambda i: (0, 0)),
            pl.BlockSpec((H, H), lambda i: (0, 0)),
        ],
        out_specs=pl.BlockSpec((OB, H), lambda i: (i, 0)),
        out_shape=jax.ShapeDtypeStruct((B // nsplit, H), jnp.float32),
    )(h2part, hsel, hsel, G1T, W1s, W1n, W2s, W2n)


def kernel(seeds, nbr1, nbr2, emb, W1s, W1n, W2s, W2n):
    B, F1 = nbr1.shape
    F2 = nbr2.shape[1]
    D = emb.shape[1]
    H = W1s.shape[1]

    nbr2f = nbr2.reshape(-1)
    sel = jnp.concatenate([nbr1.reshape(-1), seeds])

    parts = []
    hsel = None
    for s in range(_NSPLIT):
        if s == 0:
            h2p, hsel = _sc_gather(emb, nbr2f, sel, s, _NSPLIT)
        else:
            h2p = _sc_gather(emb, nbr2f, None, s, _NSPLIT)
        parts.append(h2p)

    BLK = 512
    OB = BLK // F1
    G1T = (jnp.arange(OB, dtype=jnp.int32)[:, None]
           == (jnp.arange(BLK, dtype=jnp.int32)[None, :] // F1)
           ).astype(jnp.float32) * (1.0 / F1)

    outs = [_tc_dense(parts[s], hsel, G1T, W1s, W1n, W2s, W2n, s, _NSPLIT,
                      B, F1, F2, D, H)
            for s in range(_NSPLIT)]
    return jnp.concatenate(outs, axis=0)


# speedup vs baseline: 1.0611x; 1.0611x over previous
import functools

import jax
import jax.numpy as jnp
from jax import lax
from jax.experimental import pallas as pl
from jax.experimental.pallas import tpu as pltpu
from jax.experimental.pallas import tpu_sc as plsc

_NW = 32
_CH = 128
_SCH = 32
_NB = 4
_NSPLIT = 1


def _sc_gather(emb, nbr2f, sel, split, nsplit):
    V, D = emb.shape
    N2 = nbr2f.shape[0]
    NP = N2 // nsplit
    n2_w = NP // _NW
    n2_ch = n2_w // _CH
    gbase = split * NP

    out_type = [jax.ShapeDtypeStruct((NP, D), emb.dtype)]
    scratch = [
        pltpu.VMEM((_NB, _CH), jnp.int32),
        pltpu.VMEM((_NB, _CH, D), emb.dtype),
        pltpu.SemaphoreType.DMA((_NB,)),
        pltpu.SemaphoreType.DMA((_NB,)),
    ]
    if sel is not None:
        NS = sel.shape[0]
        sel_w = NS // _NW
        sel_ch = sel_w // _SCH
        out_type.append(jax.ShapeDtypeStruct((NS, D), emb.dtype))
        scratch += [
            pltpu.VMEM((2, _SCH), jnp.int32),
            pltpu.VMEM((2, _SCH, D), emb.dtype),
            pltpu.SemaphoreType.DMA((2,)),
            pltpu.SemaphoreType.DMA((2,)),
        ]

    mesh = plsc.VectorSubcoreMesh(core_axis_name="c", subcore_axis_name="s")

    @functools.partial(pl.kernel, mesh=mesh, out_type=out_type,
                       scratch_types=scratch)
    def k(*refs):
        if sel is not None:
            (emb_hbm, n2_hbm, sel_hbm, h2_hbm, hs_hbm,
             idx_v, rows_v, gs, ws, sidx_v, srows_v, sgs, sws) = refs
        else:
            emb_hbm, n2_hbm, h2_hbm, idx_v, rows_v, gs, ws = refs

        wid = lax.axis_index("s") * 2 + lax.axis_index("c")
        base2 = wid * n2_w

        @pl.loop(0, n2_ch, step=_NB)
        def _(c):
            for b in range(_NB):
                cc = c + b

                @pl.when(cc >= _NB)
                def _():
                    pltpu.make_async_copy(
                        rows_v.at[b], h2_hbm.at[pl.ds(0, _CH)], ws.at[b]
                    ).wait()

                pltpu.sync_copy(
                    n2_hbm.at[pl.ds(gbase + base2 + cc * _CH, _CH)],
                    idx_v.at[b])
                pltpu.async_copy(emb_hbm.at[idx_v.at[b]], rows_v.at[b],
                                 gs.at[b])
                pb = (b - 1) % _NB

                @pl.when(cc >= 1)
                def _():
                    pltpu.make_async_copy(
                        emb_hbm.at[idx_v.at[pb]], rows_v.at[pb], gs.at[pb]
                    ).wait()
                    pltpu.async_copy(rows_v.at[pb],
                                     h2_hbm.at[pl.ds(base2 + cc * _CH - _CH,
                                                     _CH)],
                                     ws.at[pb])

        lb = (n2_ch - 1) % _NB
        pltpu.make_async_copy(emb_hbm.at[idx_v.at[lb]], rows_v.at[lb],
                              gs.at[lb]).wait()
        pltpu.async_copy(rows_v.at[lb],
                         h2_hbm.at[pl.ds(base2 + (n2_ch - 1) * _CH, _CH)],
                         ws.at[lb])
        for b in range(_NB):
            pltpu.make_async_copy(rows_v.at[b], h2_hbm.at[pl.ds(0, _CH)],
                                  ws.at[b]).wait()

        if sel is not None:
            bases = wid * sel_w

            @pl.loop(0, sel_ch, step=2)
            def _(c):
                for b in range(2):
                    cc = c + b

                    @pl.when(cc >= 2)
                    def _():
                        pltpu.make_async_copy(
                            srows_v.at[b], hs_hbm.at[pl.ds(0, _SCH)],
                            sws.at[b]).wait()

                    pltpu.sync_copy(sel_hbm.at[pl.ds(bases + cc * _SCH,
                                                     _SCH)],
                                    sidx_v.at[b])
                    pltpu.async_copy(emb_hbm.at[sidx_v.at[b]], srows_v.at[b],
                                     sgs.at[b])
                    pb = 1 - b

                    @pl.when(cc >= 1)
                    def _():
                        pltpu.make_async_copy(
                            emb_hbm.at[sidx_v.at[pb]], srows_v.at[pb],
                            sgs.at[pb]).wait()
                        pltpu.async_copy(
                            srows_v.at[pb],
                            hs_hbm.at[pl.ds(bases + cc * _SCH - _SCH, _SCH)],
                            sws.at[pb])

            lbs = (sel_ch - 1) % 2
            pltpu.make_async_copy(emb_hbm.at[sidx_v.at[lbs]],
                                  srows_v.at[lbs], sgs.at[lbs]).wait()
            pltpu.async_copy(srows_v.at[lbs],
                             hs_hbm.at[pl.ds(bases + (sel_ch - 1) * _SCH,
                                             _SCH)],
                             sws.at[lbs])
            for b in range(2):
                pltpu.make_async_copy(srows_v.at[b],
                                      hs_hbm.at[pl.ds(0, _SCH)],
                                      sws.at[b]).wait()

    if sel is not None:
        return k(emb, nbr2f, sel)
    res = k(emb, nbr2f)
    return res[0] if isinstance(res, (list, tuple)) else res


def _tc_dense(h2part, hsel, G1T, W1s, W1n, W2s, W2n, split, nsplit,
              B, F1, F2, D, H):
    BLK = 2048
    OB = BLK // F1
    nblk = (B * F1) // BLK // nsplit
    h1_blk0 = split * nblk
    h0_blk0 = (B * F1) // OB + split * ((B // nsplit) // OB)

    def body(h2v_ref, hs_ref, h0_ref, g_ref, w1s_ref, w1n_ref, w2s_ref,
             w2n_ref, o_ref):
        h2m = jnp.sum(h2v_ref[...].reshape(BLK, F2, D), axis=1) * (1.0 / F2)
        h1 = jnp.maximum(
            jnp.dot(hs_ref[...], w1s_ref[...],
                    preferred_element_type=jnp.float32)
            + jnp.dot(h2m, w1n_ref[...], preferred_element_type=jnp.float32),
            0.0)
        h1m = jnp.dot(g_ref[...], h1, preferred_element_type=jnp.float32)
        o_ref[...] = jnp.maximum(
            jnp.dot(h0_ref[...], w2s_ref[...],
                    preferred_element_type=jnp.float32)
            + jnp.dot(h1m, w2n_ref[...], preferred_element_type=jnp.float32),
            0.0)

    return pl.pallas_call(
        body,
        grid=(nblk,),
        in_specs=[
            pl.BlockSpec((BLK * F2, D), lambda i: (i, 0)),
            pl.BlockSpec((BLK, D), lambda i: (i + h1_blk0, 0)),
            pl.BlockSpec((OB, D), lambda i: (i + h0_blk0, 0)),
            pl.BlockSpec((OB, BLK), lambda i: (0, 0)),
            pl.BlockSpec((D, H), lambda i: (0, 0)),
            pl.BlockSpec((D, H), lambda i: (0, 0)),
            pl.BlockSpec((D, H), lambda i: (0, 0)),
            pl.BlockSpec((H, H), lambda i: (0, 0)),
        ],
        out_specs=pl.BlockSpec((OB, H), lambda i: (i, 0)),
        out_shape=jax.ShapeDtypeStruct((B // nsplit, H), jnp.float32),
    )(h2part, hsel, hsel, G1T, W1s, W1n, W2s, W2n)


def kernel(seeds, nbr1, nbr2, emb, W1s, W1n, W2s, W2n):
    B, F1 = nbr1.shape
    F2 = nbr2.shape[1]
    D = emb.shape[1]
    H = W1s.shape[1]

    nbr2f = nbr2.reshape(-1)
    sel = jnp.concatenate([nbr1.reshape(-1), seeds])

    parts = []
    hsel = None
    for s in range(_NSPLIT):
        if s == 0:
            h2p, hsel = _sc_gather(emb, nbr2f, sel, s, _NSPLIT)
        else:
            h2p = _sc_gather(emb, nbr2f, None, s, _NSPLIT)
        parts.append(h2p)

    BLK = 2048
    OB = BLK // F1
    G1T = (jnp.arange(OB, dtype=jnp.int32)[:, None]
           == (jnp.arange(BLK, dtype=jnp.int32)[None, :] // F1)
           ).astype(jnp.float32) * (1.0 / F1)

    outs = [_tc_dense(parts[s], hsel, G1T, W1s, W1n, W2s, W2n, s, _NSPLIT,
                      B, F1, F2, D, H)
            for s in range(_NSPLIT)]
    return jnp.concatenate(outs, axis=0)
